# baseline (speedup 1.0000x reference)
import jax
import jax.numpy as jnp
from jax.experimental import pallas as pl
from jax.experimental.pallas import tpu as pltpu

INPUT_DIM = 832
HIDDEN1 = 256
HIDDEN2 = 32
BATCH = 16384
BM = 2048


def _dot_t(a, b):
    return jax.lax.dot_general(
        a, b, (((1,), (1,)), ((), ())), preferred_element_type=jnp.float32)


def _mlp_block(x_ref, w1_ref, b1_ref, w2_ref, b2_ref, w3_ref, b3_ref, out_ref):
    x = x_ref[...]
    h = jnp.maximum(_dot_t(x, w1_ref[...]) + b1_ref[...], 0.0)
    h = jnp.maximum(_dot_t(h, w2_ref[...]) + b2_ref[...], 0.0)
    out = jnp.sum(h * w3_ref[...], axis=1, keepdims=True) + b3_ref[0, 0]
    out_ref[...] = out


def kernel(x, w1, b1, w2, b2, w3, b3):
    b1r = b1.reshape(1, HIDDEN1)
    b2r = b2.reshape(1, HIDDEN2)
    b3r = b3.reshape(1, 1)

    grid = (BATCH // BM,)
    const = lambda i: (0, 0)
    return pl.pallas_call(
        _mlp_block,
        grid=grid,
        in_specs=[
            pl.BlockSpec((BM, INPUT_DIM), lambda i: (i, 0)),
            pl.BlockSpec((HIDDEN1, INPUT_DIM), const),
            pl.BlockSpec((1, HIDDEN1), const),
            pl.BlockSpec((HIDDEN2, HIDDEN1), const),
            pl.BlockSpec((1, HIDDEN2), const),
            pl.BlockSpec((1, HIDDEN2), const),
            pl.BlockSpec((1, 1), const),
        ],
        out_specs=pl.BlockSpec((BM, 1), lambda i: (i, 0)),
        out_shape=jax.ShapeDtypeStruct((BATCH, 1), jnp.float32),
    )(x, w1, b1r, w2, b2r, w3, b3r)


# speedup vs baseline: 3.2943x; 3.2943x over previous
import jax
import jax.numpy as jnp
from jax.experimental import pallas as pl

INPUT_DIM = 832
HIDDEN1 = 256
HIDDEN2 = 32
BATCH = 16384
BN = 2048


def _mlp_block(xt_ref, w1t_ref, b1_ref, w2_ref, b2_ref, w3_ref, b3_ref, out_ref):
    xt = xt_ref[...]
    h = jax.lax.dot_general(
        w1t_ref[...], xt, (((0,), (0,)), ((), ())),
        preferred_element_type=jnp.float32)
    h = jnp.maximum(h + b1_ref[...], 0.0)
    h = jax.lax.dot_general(
        w2_ref[...], h, (((1,), (0,)), ((), ())),
        preferred_element_type=jnp.float32)
    h = jnp.maximum(h + b2_ref[...], 0.0)
    out = jnp.sum(h * w3_ref[...], axis=0, keepdims=True) + b3_ref[0, 0]
    out_ref[...] = out


def kernel(x, w1, b1, w2, b2, w3, b3):
    xt = x.T
    w1t = w1.T
    b1c = b1.reshape(HIDDEN1, 1)
    b2c = b2.reshape(HIDDEN2, 1)
    w3c = w3.reshape(HIDDEN2, 1)
    b3r = b3.reshape(1, 1)

    grid = (BATCH // BN,)
    const = lambda i: (0, 0)
    outt = pl.pallas_call(
        _mlp_block,
        grid=grid,
        in_specs=[
            pl.BlockSpec((INPUT_DIM, BN), lambda i: (0, i)),
            pl.BlockSpec((INPUT_DIM, HIDDEN1), const),
            pl.BlockSpec((HIDDEN1, 1), const),
            pl.BlockSpec((HIDDEN2, HIDDEN1), const),
            pl.BlockSpec((HIDDEN2, 1), const),
            pl.BlockSpec((HIDDEN2, 1), const),
            pl.BlockSpec((1, 1), const),
        ],
        out_specs=pl.BlockSpec((1, BN), lambda i: (0, i)),
        out_shape=jax.ShapeDtypeStruct((1, BATCH), jnp.float32),
    )(xt, w1t, b1c, w2, b2c, w3c, b3r)
    return outt.reshape(BATCH, 1)


# speedup vs baseline: 3.3191x; 1.0075x over previous
import jax
import jax.numpy as jnp
from jax.experimental import pallas as pl

INPUT_DIM = 832
HIDDEN1 = 256
HIDDEN2 = 32
BATCH = 16384
BN = 2048


def _mlp_block(xt_ref, w1t_ref, b1_ref, w2_ref, b2_ref, w3_ref, b3_ref, out_ref):
    xt = xt_ref[...].astype(jnp.bfloat16)
    h = jax.lax.dot_general(
        w1t_ref[...].astype(jnp.bfloat16), xt, (((0,), (0,)), ((), ())),
        preferred_element_type=jnp.float32)
    h = jnp.maximum(h + b1_ref[...], 0.0)
    h = jax.lax.dot_general(
        w2_ref[...].astype(jnp.bfloat16), h.astype(jnp.bfloat16),
        (((1,), (0,)), ((), ())),
        preferred_element_type=jnp.float32)
    h = jnp.maximum(h + b2_ref[...], 0.0)
    out = jnp.sum(h * w3_ref[...], axis=0, keepdims=True) + b3_ref[0, 0]
    out_ref[...] = out


def kernel(x, w1, b1, w2, b2, w3, b3):
    xt = x.T
    w1t = w1.T
    b1c = b1.reshape(HIDDEN1, 1)
    b2c = b2.reshape(HIDDEN2, 1)
    w3c = w3.reshape(HIDDEN2, 1)
    b3r = b3.reshape(1, 1)

    grid = (BATCH // BN,)
    const = lambda i: (0, 0)
    outt = pl.pallas_call(
        _mlp_block,
        grid=grid,
        in_specs=[
            pl.BlockSpec((INPUT_DIM, BN), lambda i: (0, i)),
            pl.BlockSpec((INPUT_DIM, HIDDEN1), const),
            pl.BlockSpec((HIDDEN1, 1), const),
            pl.BlockSpec((HIDDEN2, HIDDEN1), const),
            pl.BlockSpec((HIDDEN2, 1), const),
            pl.BlockSpec((HIDDEN2, 1), const),
            pl.BlockSpec((1, 1), const),
        ],
        out_specs=pl.BlockSpec((1, BN), lambda i: (0, i)),
        out_shape=jax.ShapeDtypeStruct((1, BATCH), jnp.float32),
    )(xt, w1t, b1c, w2, b2c, w3c, b3r)
    return outt.reshape(BATCH, 1)


# speedup vs baseline: 3.3239x; 1.0015x over previous
import jax
import jax.numpy as jnp
from jax.experimental import pallas as pl

INPUT_DIM = 832
HIDDEN1 = 256
HIDDEN2 = 32
BATCH = 16384
BN = 4096


def _mlp_block(xt_ref, w1t_ref, b1_ref, w2_ref, b2_ref, w3_ref, b3_ref, out_ref):
    xt = xt_ref[...].astype(jnp.bfloat16)
    h = jax.lax.dot_general(
        w1t_ref[...].astype(jnp.bfloat16), xt, (((0,), (0,)), ((), ())),
        preferred_element_type=jnp.float32)
    h = jnp.maximum(h + b1_ref[...], 0.0)
    h = jax.lax.dot_general(
        w2_ref[...].astype(jnp.bfloat16), h.astype(jnp.bfloat16),
        (((1,), (0,)), ((), ())),
        preferred_element_type=jnp.float32)
    h = jnp.maximum(h + b2_ref[...], 0.0)
    out = jnp.sum(h * w3_ref[...], axis=0, keepdims=True) + b3_ref[0, 0]
    out_ref[...] = out


def kernel(x, w1, b1, w2, b2, w3, b3):
    xt = x.T
    w1t = w1.T
    b1c = b1.reshape(HIDDEN1, 1)
    b2c = b2.reshape(HIDDEN2, 1)
    w3c = w3.reshape(HIDDEN2, 1)
    b3r = b3.reshape(1, 1)

    grid = (BATCH // BN,)
    const = lambda i: (0, 0)
    outt = pl.pallas_call(
        _mlp_block,
        grid=grid,
        in_specs=[
            pl.BlockSpec((INPUT_DIM, BN), lambda i: (0, i)),
            pl.BlockSpec((INPUT_DIM, HIDDEN1), const),
            pl.BlockSpec((HIDDEN1, 1), const),
            pl.BlockSpec((HIDDEN2, HIDDEN1), const),
            pl.BlockSpec((HIDDEN2, 1), const),
            pl.BlockSpec((HIDDEN2, 1), const),
            pl.BlockSpec((1, 1), const),
        ],
        out_specs=pl.BlockSpec((1, BN), lambda i: (0, i)),
        out_shape=jax.ShapeDtypeStruct((1, BATCH), jnp.float32),
    )(xt, w1t, b1c, w2, b2c, w3c, b3r)
    return outt.reshape(BATCH, 1)
